# baseline (speedup 1.0000x reference)
import functools

import jax
import jax.numpy as jnp
from jax import lax
from jax.experimental import pallas as pl
from jax.experimental.pallas import tpu as pltpu
from jax.experimental.pallas import tpu_sc as plsc

_K = 64
_C = 256
_ITERS = 100
_INF = 1e12
_B = 8


def _ot_kernel(rf_ref, sf_ref, rm_ref, sm_ref, alpha_ref, out_ref):
    B = rf_ref.shape[0]
    rf = rf_ref[...]
    sf = sf_ref[...]
    scores = jax.lax.dot_general(
        rf, sf, (((2,), (2,)), ((0,), (0,))),
        preferred_element_type=jnp.float32) * (1.0 / 16.0)
    rm = rm_ref[...]
    sm = sm_ref[...]
    alpha = alpha_ref[0, 0]

    pad_col = jnp.full((B, _K, 1), alpha, jnp.float32)
    pad_row = jnp.full((B, 1, _K + 1), alpha, jnp.float32)
    padded = jnp.concatenate([jnp.concatenate([scores, pad_col], 2), pad_row], 1)
    ones = jnp.ones((B, 1), jnp.float32)
    prm = jnp.concatenate([rm, ones], 1)
    pcm = jnp.concatenate([sm, ones], 1)
    smask = prm[:, :, None] * pcm[:, None, :]
    padded = jnp.where(smask > 0.5, padded, -_INF)

    nvr = rm.sum(1)
    nvc = sm.sum(1)
    norm = -jnp.log(jnp.maximum(nvr + nvc, 1.0))
    log_mu = jnp.concatenate(
        [jnp.where(rm > 0.5, norm[:, None], -_INF),
         (jnp.log(jnp.maximum(nvc, 1.0)) + norm)[:, None]], 1)
    log_nu = jnp.concatenate(
        [jnp.where(sm > 0.5, norm[:, None], -_INF),
         (jnp.log(jnp.maximum(nvr, 1.0)) + norm)[:, None]], 1)

    E = jnp.exp(padded)
    ET = jnp.swapaxes(E, 1, 2)
    validr = prm > 0.5
    validc = pcm > 0.5

    def body(_, uv):
        u, v = uv
        r = jax.lax.dot_general(E, jnp.exp(v), (((2,), (1,)), ((0,), (0,))),
                                preferred_element_type=jnp.float32)
        u = jnp.where(validr, log_mu - jnp.log(r), 0.0)
        c = jax.lax.dot_general(ET, jnp.exp(u), (((2,), (1,)), ((0,), (0,))),
                                preferred_element_type=jnp.float32)
        v = jnp.where(validc, log_nu - jnp.log(c), 0.0)
        return (u, v)

    u, v = jax.lax.fori_loop(
        0, _ITERS, body,
        (jnp.zeros((B, _K + 1), jnp.float32), jnp.zeros((B, _K + 1), jnp.float32)))
    out_ref[...] = padded + u[:, :, None] + v[:, None, :] - norm[:, None, None]


def _ot_sinkhorn(ref_ck_feats, src_ck_feats, ref_ck_masks, src_ck_masks, alpha):
    alpha_arr = jnp.reshape(alpha, (1, 1)).astype(jnp.float32)
    grid = (_C // _B,)
    return pl.pallas_call(
        _ot_kernel,
        grid=grid,
        in_specs=[
            pl.BlockSpec((_B, _K, 256), lambda b: (b, 0, 0)),
            pl.BlockSpec((_B, _K, 256), lambda b: (b, 0, 0)),
            pl.BlockSpec((_B, _K), lambda b: (b, 0)),
            pl.BlockSpec((_B, _K), lambda b: (b, 0)),
            pl.BlockSpec((1, 1), lambda b: (0, 0)),
        ],
        out_specs=pl.BlockSpec((_B, _K + 1, _K + 1), lambda b: (b, 0, 0)),
        out_shape=jax.ShapeDtypeStruct((_C, _K + 1, _K + 1), jnp.float32),
    )(ref_ck_feats, src_ck_feats,
      ref_ck_masks.astype(jnp.float32), src_ck_masks.astype(jnp.float32),
      alpha_arr)


_NW = 32
_ROWS = _C * _K
_RPW = _ROWS // _NW
_CPW = _C // _NW


def _sc_gather_kernel(rt_hbm, st_hbm, ri_hbm, si_hbm,
                      orf, osf,
                      riv, siv, buf0, buf1, sem0, sem1):
    wid = lax.axis_index("s") * 2 + lax.axis_index("c")
    cbase = wid * _CPW
    rbase = wid * _RPW
    pltpu.sync_copy(ri_hbm.at[pl.ds(cbase, _CPW)], riv)
    pltpu.sync_copy(si_hbm.at[pl.ds(cbase, _CPW)], siv)

    def chunk(j, _):
        cp0 = pltpu.make_async_copy(rt_hbm.at[riv.at[j]], buf0, sem0)
        cp1 = pltpu.make_async_copy(st_hbm.at[siv.at[j]], buf1, sem1)
        cp0.start(); cp1.start()
        cp0.wait()
        pltpu.sync_copy(buf0, orf.at[pl.ds(rbase + j * _K, _K)])
        cp1.wait()
        pltpu.sync_copy(buf1, osf.at[pl.ds(rbase + j * _K, _K)])
        return ()

    lax.fori_loop(0, _CPW, chunk, (), unroll=False)


def _sc_gather(ref_table, src_table, ref_idx, src_idx):
    mesh = plsc.VectorSubcoreMesh(core_axis_name="c", subcore_axis_name="s")
    d = ref_table.shape[1]
    f = pl.kernel(
        _sc_gather_kernel,
        mesh=mesh,
        out_type=[
            jax.ShapeDtypeStruct((_ROWS, d), jnp.float32),
            jax.ShapeDtypeStruct((_ROWS, d), jnp.float32),
        ],
        scratch_types=[
            pltpu.VMEM((_CPW, _K), jnp.int32),
            pltpu.VMEM((_CPW, _K), jnp.int32),
            pltpu.VMEM((_K, d), jnp.float32),
            pltpu.VMEM((_K, d), jnp.float32),
            pltpu.SemaphoreType.DMA,
            pltpu.SemaphoreType.DMA,
        ],
    )
    return f(ref_table, src_table, ref_idx, src_idx)


def _sq_dist_k(a, b):
    return jnp.maximum(
        (a * a).sum(-1)[:, None] + (b * b).sum(-1)[None, :] - 2.0 * (a @ b.T), 0.0)


_NB = 8
_NC_NODES = 512
_NF_PTS = 16384
_BIG_I = 2 ** 24


def _knn_kernel(dT_ref, knn_ref, p2n_ref, runv_ref, runi_ref):
    blk = pl.program_id(0)
    d0 = dT_ref[...]
    lane = jax.lax.broadcasted_iota(jnp.int32, (_NB, _NF_PTS), 1)

    sub = jax.lax.broadcasted_iota(jnp.int32, (_NB, _NF_PTS), 0)
    bm = jnp.min(d0, axis=0, keepdims=True)
    bi = (jnp.min(jnp.where(d0 == bm, sub, _BIG_I), axis=0, keepdims=True)
          + 8 * blk)

    @pl.when(blk == 0)
    def _():
        runv_ref[...] = bm
        runi_ref[...] = bi

    @pl.when(blk > 0)
    def _():
        rv = runv_ref[...]
        take = bm < rv
        runv_ref[...] = jnp.where(take, bm, rv)
        runi_ref[...] = jnp.where(take, bi, runi_ref[...])

    @pl.when(blk == pl.num_programs(0) - 1)
    def _():
        p2n_ref[...] = runi_ref[...]

    lane64 = jax.lax.broadcasted_iota(jnp.int32, (_NB, _K), 1)

    def step(j, carry):
        x, acc = carry
        v, i = x, lane
        w = _NF_PTS
        while w > 128:
            h = w // 2
            va, vb = v[:, :h], v[:, h:]
            ia, ib = i[:, :h], i[:, h:]
            t = vb < va
            v = jnp.where(t, vb, va)
            i = jnp.where(t, ib, ia)
            w = h
        m = jnp.min(v, axis=1, keepdims=True)
        ii = jnp.min(jnp.where(v == m, i, _BIG_I), axis=1, keepdims=True)
        acc = jnp.where(lane64 == j, ii, acc)
        return jnp.where(lane == ii, jnp.float32(_INF), x), acc

    _, acc = jax.lax.fori_loop(
        0, _K, step, (d0, jnp.zeros((_NB, _K), jnp.int32)), unroll=4)
    knn_ref[...] = acc


def _knn_topk(dist2T):
    grid = (_NC_NODES // _NB,)
    knn, p2n = pl.pallas_call(
        _knn_kernel,
        grid=grid,
        in_specs=[pl.BlockSpec((_NB, _NF_PTS), lambda b: (b, 0))],
        out_specs=[pl.BlockSpec((_NB, _K), lambda b: (b, 0)),
                   pl.BlockSpec((1, _NF_PTS), lambda b: (0, 0))],
        out_shape=[jax.ShapeDtypeStruct((_NC_NODES, _K), jnp.int32),
                   jax.ShapeDtypeStruct((1, _NF_PTS), jnp.int32)],
        scratch_shapes=[pltpu.VMEM((1, _NF_PTS), jnp.float32),
                        pltpu.VMEM((1, _NF_PTS), jnp.int32)],
    )(dist2T)
    return knn, p2n.reshape(_NF_PTS)


def _partition(points_f, points_c, k):
    dist2 = _sq_dist_k(points_f, points_c)
    knn_indices, point_to_node = _knn_topk(dist2.T)
    counts = jnp.zeros((points_c.shape[0],), jnp.int32).at[point_to_node].add(1)
    node_masks = counts > 0
    knn_masks = point_to_node[knn_indices] == jnp.arange(points_c.shape[0])[:, None]
    knn_indices = jnp.where(knn_masks, knn_indices, points_f.shape[0])
    return node_masks, knn_indices, knn_masks


def kernel(ref_points_f, src_points_f, ref_points_c, src_points_c,
           ref_feats_f, src_feats_f, ref_feats_c, src_feats_c, alpha):
    k = _K
    ref_node_masks, ref_knn_idx, ref_knn_masks = _partition(ref_points_f, ref_points_c, k)
    src_node_masks, src_knn_idx, src_knn_masks = _partition(src_points_f, src_points_c, k)

    ref_padded_points = jnp.concatenate([ref_points_f, jnp.zeros((1, 3), jnp.float32)], axis=0)
    src_padded_points = jnp.concatenate([src_points_f, jnp.zeros((1, 3), jnp.float32)], axis=0)

    rfn = ref_feats_c / (jnp.linalg.norm(ref_feats_c, axis=1, keepdims=True) + 1e-12)
    sfn = src_feats_c / (jnp.linalg.norm(src_feats_c, axis=1, keepdims=True) + 1e-12)
    dist = jnp.maximum(2.0 - 2.0 * (rfn @ sfn.T), 0.0)
    scores = jnp.exp(-dist)
    scores = (scores / scores.sum(1, keepdims=True)) * (scores / scores.sum(0, keepdims=True))
    pair_mask = ref_node_masks[:, None] & src_node_masks[None, :]
    scores = jnp.where(pair_mask, scores, 0.0)
    node_corr_scores, corr_idx = jax.lax.top_k(scores.reshape(-1), _C)
    Mc = src_feats_c.shape[0]
    ref_corr = corr_idx // Mc
    src_corr = corr_idx % Mc

    ref_ck_idx = ref_knn_idx[ref_corr]
    src_ck_idx = src_knn_idx[src_corr]
    ref_ck_masks = ref_knn_masks[ref_corr]
    src_ck_masks = src_knn_masks[src_corr]

    d = ref_feats_f.shape[1]
    ref_table = jnp.concatenate(
        [ref_padded_feats := jnp.concatenate(
            [ref_feats_f, jnp.zeros((1, d), jnp.float32)], axis=0),
         jnp.pad(ref_padded_points, ((0, 0), (0, 125)))], axis=1)
    src_table = jnp.concatenate(
        [src_padded_feats := jnp.concatenate(
            [src_feats_f, jnp.zeros((1, d), jnp.float32)], axis=0),
         jnp.pad(src_padded_points, ((0, 0), (0, 125)))], axis=1)
    del ref_padded_feats, src_padded_feats

    rg, sg = _sc_gather(ref_table, src_table, ref_ck_idx, src_ck_idx)
    ref_ck_feats = rg[:, :d].reshape(_C, _K, d)
    src_ck_feats = sg[:, :d].reshape(_C, _K, d)
    ref_ck_points = rg[:, d:d + 3].reshape(_C, _K, 3)
    src_ck_points = sg[:, d:d + 3].reshape(_C, _K, 3)

    matching_scores = _ot_sinkhorn(ref_ck_feats, src_ck_feats, ref_ck_masks, src_ck_masks, alpha)
    return matching_scores, node_corr_scores, ref_corr, src_corr, ref_ck_points, src_ck_points


# speedup vs baseline: 1.1072x; 1.1072x over previous
import functools

import jax
import jax.numpy as jnp
from jax import lax
from jax.experimental import pallas as pl
from jax.experimental.pallas import tpu as pltpu
from jax.experimental.pallas import tpu_sc as plsc

_K = 64
_C = 256
_ITERS = 100
_INF = 1e12
_B = 8


def _ot_kernel(rf_ref, sf_ref, rm_ref, sm_ref, alpha_ref, out_ref):
    B = rf_ref.shape[0]
    rf = rf_ref[...]
    sf = sf_ref[...]
    scores = jax.lax.dot_general(
        rf, sf, (((2,), (2,)), ((0,), (0,))),
        preferred_element_type=jnp.float32) * (1.0 / 16.0)
    rm = rm_ref[...]
    sm = sm_ref[...]
    alpha = alpha_ref[0, 0]

    pad_col = jnp.full((B, _K, 1), alpha, jnp.float32)
    pad_row = jnp.full((B, 1, _K + 1), alpha, jnp.float32)
    padded = jnp.concatenate([jnp.concatenate([scores, pad_col], 2), pad_row], 1)
    ones = jnp.ones((B, 1), jnp.float32)
    prm = jnp.concatenate([rm, ones], 1)
    pcm = jnp.concatenate([sm, ones], 1)
    smask = prm[:, :, None] * pcm[:, None, :]
    padded = jnp.where(smask > 0.5, padded, -_INF)

    nvr = rm.sum(1)
    nvc = sm.sum(1)
    norm = -jnp.log(jnp.maximum(nvr + nvc, 1.0))
    log_mu = jnp.concatenate(
        [jnp.where(rm > 0.5, norm[:, None], -_INF),
         (jnp.log(jnp.maximum(nvc, 1.0)) + norm)[:, None]], 1)
    log_nu = jnp.concatenate(
        [jnp.where(sm > 0.5, norm[:, None], -_INF),
         (jnp.log(jnp.maximum(nvr, 1.0)) + norm)[:, None]], 1)

    E = jnp.exp(padded)
    validr = prm > 0.5
    validc = pcm > 0.5

    def body(_, uv):
        u, v = uv
        r = (E * jnp.exp(v)[:, None, :]).sum(2)
        u = jnp.where(validr, log_mu - jnp.log(r), 0.0)
        c = (E * jnp.exp(u)[:, :, None]).sum(1)
        v = jnp.where(validc, log_nu - jnp.log(c), 0.0)
        return (u, v)

    u, v = jax.lax.fori_loop(
        0, _ITERS, body,
        (jnp.zeros((B, _K + 1), jnp.float32), jnp.zeros((B, _K + 1), jnp.float32)),
        unroll=2)
    out_ref[...] = padded + u[:, :, None] + v[:, None, :] - norm[:, None, None]


def _ot_sinkhorn(ref_ck_feats, src_ck_feats, ref_ck_masks, src_ck_masks, alpha):
    alpha_arr = jnp.reshape(alpha, (1, 1)).astype(jnp.float32)
    grid = (_C // _B,)
    return pl.pallas_call(
        _ot_kernel,
        grid=grid,
        in_specs=[
            pl.BlockSpec((_B, _K, 256), lambda b: (b, 0, 0)),
            pl.BlockSpec((_B, _K, 256), lambda b: (b, 0, 0)),
            pl.BlockSpec((_B, _K), lambda b: (b, 0)),
            pl.BlockSpec((_B, _K), lambda b: (b, 0)),
            pl.BlockSpec((1, 1), lambda b: (0, 0)),
        ],
        out_specs=pl.BlockSpec((_B, _K + 1, _K + 1), lambda b: (b, 0, 0)),
        out_shape=jax.ShapeDtypeStruct((_C, _K + 1, _K + 1), jnp.float32),
    )(ref_ck_feats, src_ck_feats,
      ref_ck_masks.astype(jnp.float32), src_ck_masks.astype(jnp.float32),
      alpha_arr)


_NW = 32
_ROWS = _C * _K
_RPW = _ROWS // _NW
_CPW = _C // _NW


def _sc_gather_kernel(rt_hbm, st_hbm, ri_hbm, si_hbm,
                      orf, osf,
                      riv, siv, buf0, buf1, sem0, sem1):
    wid = lax.axis_index("s") * 2 + lax.axis_index("c")
    cbase = wid * _CPW
    rbase = wid * _RPW
    pltpu.sync_copy(ri_hbm.at[pl.ds(cbase, _CPW)], riv)
    pltpu.sync_copy(si_hbm.at[pl.ds(cbase, _CPW)], siv)

    def chunk(j, _):
        cp0 = pltpu.make_async_copy(rt_hbm.at[riv.at[j]], buf0, sem0)
        cp1 = pltpu.make_async_copy(st_hbm.at[siv.at[j]], buf1, sem1)
        cp0.start(); cp1.start()
        cp0.wait()
        pltpu.sync_copy(buf0, orf.at[pl.ds(rbase + j * _K, _K)])
        cp1.wait()
        pltpu.sync_copy(buf1, osf.at[pl.ds(rbase + j * _K, _K)])
        return ()

    lax.fori_loop(0, _CPW, chunk, (), unroll=False)


def _sc_gather(ref_table, src_table, ref_idx, src_idx):
    mesh = plsc.VectorSubcoreMesh(core_axis_name="c", subcore_axis_name="s")
    d = ref_table.shape[1]
    f = pl.kernel(
        _sc_gather_kernel,
        mesh=mesh,
        out_type=[
            jax.ShapeDtypeStruct((_ROWS, d), jnp.float32),
            jax.ShapeDtypeStruct((_ROWS, d), jnp.float32),
        ],
        scratch_types=[
            pltpu.VMEM((_CPW, _K), jnp.int32),
            pltpu.VMEM((_CPW, _K), jnp.int32),
            pltpu.VMEM((_K, d), jnp.float32),
            pltpu.VMEM((_K, d), jnp.float32),
            pltpu.SemaphoreType.DMA,
            pltpu.SemaphoreType.DMA,
        ],
    )
    return f(ref_table, src_table, ref_idx, src_idx)


def _sq_dist_k(a, b):
    return jnp.maximum(
        (a * a).sum(-1)[:, None] + (b * b).sum(-1)[None, :] - 2.0 * (a @ b.T), 0.0)


_NB = 8
_NC_NODES = 512
_NF_PTS = 16384
_BIG_I = 2 ** 24


def _knn_kernel(dT_ref, knn_ref, p2n_ref, runv_ref, runi_ref):
    blk = pl.program_id(0)
    d0 = dT_ref[...]
    lane = jax.lax.broadcasted_iota(jnp.int32, (_NB, _NF_PTS), 1)

    sub = jax.lax.broadcasted_iota(jnp.int32, (_NB, _NF_PTS), 0)
    bm = jnp.min(d0, axis=0, keepdims=True)
    bi = (jnp.min(jnp.where(d0 == bm, sub, _BIG_I), axis=0, keepdims=True)
          + 8 * blk)

    @pl.when(blk == 0)
    def _():
        runv_ref[...] = bm
        runi_ref[...] = bi

    @pl.when(blk > 0)
    def _():
        rv = runv_ref[...]
        take = bm < rv
        runv_ref[...] = jnp.where(take, bm, rv)
        runi_ref[...] = jnp.where(take, bi, runi_ref[...])

    @pl.when(blk == pl.num_programs(0) - 1)
    def _():
        p2n_ref[...] = runi_ref[...]

    lane64 = jax.lax.broadcasted_iota(jnp.int32, (_NB, _K), 1)

    def step(j, carry):
        x, acc = carry
        v, i = x, lane
        w = _NF_PTS
        while w > 128:
            h = w // 2
            va, vb = v[:, :h], v[:, h:]
            ia, ib = i[:, :h], i[:, h:]
            t = vb < va
            v = jnp.where(t, vb, va)
            i = jnp.where(t, ib, ia)
            w = h
        m = jnp.min(v, axis=1, keepdims=True)
        ii = jnp.min(jnp.where(v == m, i, _BIG_I), axis=1, keepdims=True)
        acc = jnp.where(lane64 == j, ii, acc)
        return jnp.where(lane == ii, jnp.float32(_INF), x), acc

    _, acc = jax.lax.fori_loop(
        0, _K, step, (d0, jnp.zeros((_NB, _K), jnp.int32)), unroll=4)
    knn_ref[...] = acc


def _knn_topk(dist2T):
    grid = (_NC_NODES // _NB,)
    knn, p2n = pl.pallas_call(
        _knn_kernel,
        grid=grid,
        in_specs=[pl.BlockSpec((_NB, _NF_PTS), lambda b: (b, 0))],
        out_specs=[pl.BlockSpec((_NB, _K), lambda b: (b, 0)),
                   pl.BlockSpec((1, _NF_PTS), lambda b: (0, 0))],
        out_shape=[jax.ShapeDtypeStruct((_NC_NODES, _K), jnp.int32),
                   jax.ShapeDtypeStruct((1, _NF_PTS), jnp.int32)],
        scratch_shapes=[pltpu.VMEM((1, _NF_PTS), jnp.float32),
                        pltpu.VMEM((1, _NF_PTS), jnp.int32)],
    )(dist2T)
    return knn, p2n.reshape(_NF_PTS)


def _partition(points_f, points_c, k):
    dist2 = _sq_dist_k(points_f, points_c)
    knn_indices, point_to_node = _knn_topk(dist2.T)
    counts = jnp.zeros((points_c.shape[0],), jnp.int32).at[point_to_node].add(1)
    node_masks = counts > 0
    knn_masks = point_to_node[knn_indices] == jnp.arange(points_c.shape[0])[:, None]
    knn_indices = jnp.where(knn_masks, knn_indices, points_f.shape[0])
    return node_masks, knn_indices, knn_masks


def kernel(ref_points_f, src_points_f, ref_points_c, src_points_c,
           ref_feats_f, src_feats_f, ref_feats_c, src_feats_c, alpha):
    k = _K
    ref_node_masks, ref_knn_idx, ref_knn_masks = _partition(ref_points_f, ref_points_c, k)
    src_node_masks, src_knn_idx, src_knn_masks = _partition(src_points_f, src_points_c, k)

    ref_padded_points = jnp.concatenate([ref_points_f, jnp.zeros((1, 3), jnp.float32)], axis=0)
    src_padded_points = jnp.concatenate([src_points_f, jnp.zeros((1, 3), jnp.float32)], axis=0)

    rfn = ref_feats_c / (jnp.linalg.norm(ref_feats_c, axis=1, keepdims=True) + 1e-12)
    sfn = src_feats_c / (jnp.linalg.norm(src_feats_c, axis=1, keepdims=True) + 1e-12)
    dist = jnp.maximum(2.0 - 2.0 * (rfn @ sfn.T), 0.0)
    scores = jnp.exp(-dist)
    scores = (scores / scores.sum(1, keepdims=True)) * (scores / scores.sum(0, keepdims=True))
    pair_mask = ref_node_masks[:, None] & src_node_masks[None, :]
    scores = jnp.where(pair_mask, scores, 0.0)
    node_corr_scores, corr_idx = jax.lax.top_k(scores.reshape(-1), _C)
    Mc = src_feats_c.shape[0]
    ref_corr = corr_idx // Mc
    src_corr = corr_idx % Mc

    ref_ck_idx = ref_knn_idx[ref_corr]
    src_ck_idx = src_knn_idx[src_corr]
    ref_ck_masks = ref_knn_masks[ref_corr]
    src_ck_masks = src_knn_masks[src_corr]

    d = ref_feats_f.shape[1]
    ref_table = jnp.concatenate(
        [ref_padded_feats := jnp.concatenate(
            [ref_feats_f, jnp.zeros((1, d), jnp.float32)], axis=0),
         jnp.pad(ref_padded_points, ((0, 0), (0, 125)))], axis=1)
    src_table = jnp.concatenate(
        [src_padded_feats := jnp.concatenate(
            [src_feats_f, jnp.zeros((1, d), jnp.float32)], axis=0),
         jnp.pad(src_padded_points, ((0, 0), (0, 125)))], axis=1)
    del ref_padded_feats, src_padded_feats

    rg, sg = _sc_gather(ref_table, src_table, ref_ck_idx, src_ck_idx)
    ref_ck_feats = rg[:, :d].reshape(_C, _K, d)
    src_ck_feats = sg[:, :d].reshape(_C, _K, d)
    ref_ck_points = rg[:, d:d + 3].reshape(_C, _K, 3)
    src_ck_points = sg[:, d:d + 3].reshape(_C, _K, 3)

    matching_scores = _ot_sinkhorn(ref_ck_feats, src_ck_feats, ref_ck_masks, src_ck_masks, alpha)
    return matching_scores, node_corr_scores, ref_corr, src_corr, ref_ck_points, src_ck_points


# speedup vs baseline: 1.1160x; 1.0080x over previous
import functools

import jax
import jax.numpy as jnp
from jax import lax
from jax.experimental import pallas as pl
from jax.experimental.pallas import tpu as pltpu
from jax.experimental.pallas import tpu_sc as plsc

_K = 64
_C = 256
_ITERS = 100
_INF = 1e12
_B = 8


def _ot_kernel(rf_ref, sf_ref, rm_ref, sm_ref, alpha_ref, out_ref):
    B = rf_ref.shape[0]
    rf = rf_ref[...]
    sf = sf_ref[...]
    scores = jax.lax.dot_general(
        rf, sf, (((2,), (2,)), ((0,), (0,))),
        preferred_element_type=jnp.float32) * (1.0 / 16.0)
    rm = rm_ref[...]
    sm = sm_ref[...]
    alpha = alpha_ref[0, 0]

    pad_col = jnp.full((B, _K, 1), alpha, jnp.float32)
    pad_row = jnp.full((B, 1, _K + 1), alpha, jnp.float32)
    padded = jnp.concatenate([jnp.concatenate([scores, pad_col], 2), pad_row], 1)
    ones = jnp.ones((B, 1), jnp.float32)
    prm = jnp.concatenate([rm, ones], 1)
    pcm = jnp.concatenate([sm, ones], 1)
    smask = prm[:, :, None] * pcm[:, None, :]
    padded = jnp.where(smask > 0.5, padded, -_INF)

    nvr = rm.sum(1)
    nvc = sm.sum(1)
    norm = -jnp.log(jnp.maximum(nvr + nvc, 1.0))
    log_mu = jnp.concatenate(
        [jnp.where(rm > 0.5, norm[:, None], -_INF),
         (jnp.log(jnp.maximum(nvc, 1.0)) + norm)[:, None]], 1)
    log_nu = jnp.concatenate(
        [jnp.where(sm > 0.5, norm[:, None], -_INF),
         (jnp.log(jnp.maximum(nvr, 1.0)) + norm)[:, None]], 1)

    E = jnp.exp(padded)
    validr = prm > 0.5
    validc = pcm > 0.5

    def body(_, uv):
        u, v = uv
        r = (E * jnp.exp(v)[:, None, :]).sum(2)
        u = jnp.where(validr, log_mu - jnp.log(r), 0.0)
        c = (E * jnp.exp(u)[:, :, None]).sum(1)
        v = jnp.where(validc, log_nu - jnp.log(c), 0.0)
        return (u, v)

    u, v = jax.lax.fori_loop(
        0, _ITERS, body,
        (jnp.zeros((B, _K + 1), jnp.float32), jnp.zeros((B, _K + 1), jnp.float32)),
        unroll=2)
    out_ref[...] = padded + u[:, :, None] + v[:, None, :] - norm[:, None, None]


def _ot_sinkhorn(ref_ck_feats, src_ck_feats, ref_ck_masks, src_ck_masks, alpha):
    alpha_arr = jnp.reshape(alpha, (1, 1)).astype(jnp.float32)
    grid = (_C // _B,)
    return pl.pallas_call(
        _ot_kernel,
        grid=grid,
        in_specs=[
            pl.BlockSpec((_B, _K, 256), lambda b: (b, 0, 0)),
            pl.BlockSpec((_B, _K, 256), lambda b: (b, 0, 0)),
            pl.BlockSpec((_B, _K), lambda b: (b, 0)),
            pl.BlockSpec((_B, _K), lambda b: (b, 0)),
            pl.BlockSpec((1, 1), lambda b: (0, 0)),
        ],
        out_specs=pl.BlockSpec((_B, _K + 1, _K + 1), lambda b: (b, 0, 0)),
        out_shape=jax.ShapeDtypeStruct((_C, _K + 1, _K + 1), jnp.float32),
    )(ref_ck_feats, src_ck_feats,
      ref_ck_masks.astype(jnp.float32), src_ck_masks.astype(jnp.float32),
      alpha_arr)


_NW = 32
_ROWS = _C * _K
_RPW = _ROWS // _NW
_CPW = _C // _NW


def _sc_gather_kernel(rt_hbm, st_hbm, ri_hbm, si_hbm,
                      orf, osf,
                      riv, siv, buf0, buf1, sem0, sem1):
    wid = lax.axis_index("s") * 2 + lax.axis_index("c")
    cbase = wid * _CPW
    rbase = wid * _RPW
    pltpu.sync_copy(ri_hbm.at[pl.ds(cbase, _CPW)], riv)
    pltpu.sync_copy(si_hbm.at[pl.ds(cbase, _CPW)], siv)

    def chunk(j, _):
        cp0 = pltpu.make_async_copy(rt_hbm.at[riv.at[j]], buf0, sem0)
        cp1 = pltpu.make_async_copy(st_hbm.at[siv.at[j]], buf1, sem1)
        cp0.start(); cp1.start()
        cp0.wait()
        pltpu.sync_copy(buf0, orf.at[pl.ds(rbase + j * _K, _K)])
        cp1.wait()
        pltpu.sync_copy(buf1, osf.at[pl.ds(rbase + j * _K, _K)])
        return ()

    lax.fori_loop(0, _CPW, chunk, (), unroll=False)


def _sc_gather(ref_table, src_table, ref_idx, src_idx):
    mesh = plsc.VectorSubcoreMesh(core_axis_name="c", subcore_axis_name="s")
    d = ref_table.shape[1]
    f = pl.kernel(
        _sc_gather_kernel,
        mesh=mesh,
        out_type=[
            jax.ShapeDtypeStruct((_ROWS, d), jnp.float32),
            jax.ShapeDtypeStruct((_ROWS, d), jnp.float32),
        ],
        scratch_types=[
            pltpu.VMEM((_CPW, _K), jnp.int32),
            pltpu.VMEM((_CPW, _K), jnp.int32),
            pltpu.VMEM((_K, d), jnp.float32),
            pltpu.VMEM((_K, d), jnp.float32),
            pltpu.SemaphoreType.DMA,
            pltpu.SemaphoreType.DMA,
        ],
    )
    return f(ref_table, src_table, ref_idx, src_idx)


def _sq_dist_k(a, b):
    return jnp.maximum(
        (a * a).sum(-1)[:, None] + (b * b).sum(-1)[None, :] - 2.0 * (a @ b.T), 0.0)


_NB = 8
_NC_NODES = 512
_NF_PTS = 16384
_BIG_I = 2 ** 24


def _knn_kernel(dT_ref, knn_ref, p2n_ref, runv_ref, runi_ref):
    blk = pl.program_id(0)
    d0 = dT_ref[...]
    lane = jax.lax.broadcasted_iota(jnp.int32, (_NB, _NF_PTS), 1)

    sub = jax.lax.broadcasted_iota(jnp.int32, (_NB, _NF_PTS), 0)
    bm = jnp.min(d0, axis=0, keepdims=True)
    bi = (jnp.min(jnp.where(d0 == bm, sub, _BIG_I), axis=0, keepdims=True)
          + 8 * blk)

    @pl.when(blk == 0)
    def _():
        runv_ref[...] = bm
        runi_ref[...] = bi

    @pl.when(blk > 0)
    def _():
        rv = runv_ref[...]
        take = bm < rv
        runv_ref[...] = jnp.where(take, bm, rv)
        runi_ref[...] = jnp.where(take, bi, runi_ref[...])

    @pl.when(blk == pl.num_programs(0) - 1)
    def _():
        p2n_ref[...] = runi_ref[...]

    lane64 = jax.lax.broadcasted_iota(jnp.int32, (_NB, _K), 1)

    def step(j, carry):
        x, acc = carry
        v, i = x, lane
        w = _NF_PTS
        while w > 128:
            h = w // 2
            va, vb = v[:, :h], v[:, h:]
            ia, ib = i[:, :h], i[:, h:]
            t = vb < va
            v = jnp.where(t, vb, va)
            i = jnp.where(t, ib, ia)
            w = h
        m = jnp.min(v, axis=1, keepdims=True)
        ii = jnp.min(jnp.where(v == m, i, _BIG_I), axis=1, keepdims=True)
        acc = jnp.where(lane64 == j, ii, acc)
        return jnp.where(lane == ii, jnp.float32(_INF), x), acc

    _, acc = jax.lax.fori_loop(
        0, _K, step, (d0, jnp.zeros((_NB, _K), jnp.int32)), unroll=8)
    knn_ref[...] = acc


def _knn_topk(dist2T):
    grid = (_NC_NODES // _NB,)
    knn, p2n = pl.pallas_call(
        _knn_kernel,
        grid=grid,
        in_specs=[pl.BlockSpec((_NB, _NF_PTS), lambda b: (b, 0))],
        out_specs=[pl.BlockSpec((_NB, _K), lambda b: (b, 0)),
                   pl.BlockSpec((1, _NF_PTS), lambda b: (0, 0))],
        out_shape=[jax.ShapeDtypeStruct((_NC_NODES, _K), jnp.int32),
                   jax.ShapeDtypeStruct((1, _NF_PTS), jnp.int32)],
        scratch_shapes=[pltpu.VMEM((1, _NF_PTS), jnp.float32),
                        pltpu.VMEM((1, _NF_PTS), jnp.int32)],
    )(dist2T)
    return knn, p2n.reshape(_NF_PTS)


def _partition(points_f, points_c, k):
    dist2 = _sq_dist_k(points_f, points_c)
    knn_indices, point_to_node = _knn_topk(dist2.T)
    counts = jnp.zeros((points_c.shape[0],), jnp.int32).at[point_to_node].add(1)
    node_masks = counts > 0
    knn_masks = point_to_node[knn_indices] == jnp.arange(points_c.shape[0])[:, None]
    knn_indices = jnp.where(knn_masks, knn_indices, points_f.shape[0])
    return node_masks, knn_indices, knn_masks


def kernel(ref_points_f, src_points_f, ref_points_c, src_points_c,
           ref_feats_f, src_feats_f, ref_feats_c, src_feats_c, alpha):
    k = _K
    ref_node_masks, ref_knn_idx, ref_knn_masks = _partition(ref_points_f, ref_points_c, k)
    src_node_masks, src_knn_idx, src_knn_masks = _partition(src_points_f, src_points_c, k)

    ref_padded_points = jnp.concatenate([ref_points_f, jnp.zeros((1, 3), jnp.float32)], axis=0)
    src_padded_points = jnp.concatenate([src_points_f, jnp.zeros((1, 3), jnp.float32)], axis=0)

    rfn = ref_feats_c / (jnp.linalg.norm(ref_feats_c, axis=1, keepdims=True) + 1e-12)
    sfn = src_feats_c / (jnp.linalg.norm(src_feats_c, axis=1, keepdims=True) + 1e-12)
    dist = jnp.maximum(2.0 - 2.0 * (rfn @ sfn.T), 0.0)
    scores = jnp.exp(-dist)
    scores = (scores / scores.sum(1, keepdims=True)) * (scores / scores.sum(0, keepdims=True))
    pair_mask = ref_node_masks[:, None] & src_node_masks[None, :]
    scores = jnp.where(pair_mask, scores, 0.0)
    node_corr_scores, corr_idx = jax.lax.top_k(scores.reshape(-1), _C)
    Mc = src_feats_c.shape[0]
    ref_corr = corr_idx // Mc
    src_corr = corr_idx % Mc

    ref_ck_idx = ref_knn_idx[ref_corr]
    src_ck_idx = src_knn_idx[src_corr]
    ref_ck_masks = ref_knn_masks[ref_corr]
    src_ck_masks = src_knn_masks[src_corr]

    d = ref_feats_f.shape[1]
    ref_table = jnp.concatenate(
        [ref_padded_feats := jnp.concatenate(
            [ref_feats_f, jnp.zeros((1, d), jnp.float32)], axis=0),
         jnp.pad(ref_padded_points, ((0, 0), (0, 125)))], axis=1)
    src_table = jnp.concatenate(
        [src_padded_feats := jnp.concatenate(
            [src_feats_f, jnp.zeros((1, d), jnp.float32)], axis=0),
         jnp.pad(src_padded_points, ((0, 0), (0, 125)))], axis=1)
    del ref_padded_feats, src_padded_feats

    rg, sg = _sc_gather(ref_table, src_table, ref_ck_idx, src_ck_idx)
    ref_ck_feats = rg[:, :d].reshape(_C, _K, d)
    src_ck_feats = sg[:, :d].reshape(_C, _K, d)
    ref_ck_points = rg[:, d:d + 3].reshape(_C, _K, 3)
    src_ck_points = sg[:, d:d + 3].reshape(_C, _K, 3)

    matching_scores = _ot_sinkhorn(ref_ck_feats, src_ck_feats, ref_ck_masks, src_ck_masks, alpha)
    return matching_scores, node_corr_scores, ref_corr, src_corr, ref_ck_points, src_ck_points
